# Initial kernel scaffold; baseline (speedup 1.0000x reference)
import jax
import jax.numpy as jnp
from jax.experimental import pallas as pl


def kernel(x, table, W, b):
    raise NotImplementedError("write your pallas kernel here")


# speedup vs baseline: 3.2098x; 3.2098x over previous
import functools

import jax
import jax.numpy as jnp
from jax import lax
from jax.experimental import pallas as pl
from jax.experimental.pallas import tpu as pltpu
from jax.experimental.pallas import tpu_sc as plsc

_NUM_CORES = 2
_NUM_SUBCORES = 16
_LANES = 16
_NUM_WORKERS = _NUM_CORES * _NUM_SUBCORES


def _project_body(t_ref, w_ref, b_ref, p_ref, *, inv_l):
    t = t_ref[...]
    w = w_ref[...]
    s = jnp.sum(t * w, axis=1)
    p_ref[...] = (s + b_ref[0, 0]) * inv_l


def _project_table(table, w_row, b2, hist_len):
    v_rows, emb = table.shape
    blk = 8192
    n_blk = (v_rows + blk - 1) // blk
    return pl.pallas_call(
        functools.partial(_project_body, inv_l=1.0 / float(hist_len)),
        grid=(n_blk,),
        in_specs=[
            pl.BlockSpec((blk, emb), lambda i: (i, 0)),
            pl.BlockSpec((1, emb), lambda i: (0, 0)),
            pl.BlockSpec(memory_space=pltpu.SMEM),
        ],
        out_specs=pl.BlockSpec((blk,), lambda i: (i,)),
        out_shape=jax.ShapeDtypeStruct((v_rows,), jnp.float32),
    )(table, w_row, b2)


def _make_gather_kernel(hist_len, batch, chunk_cols, chunks_per_worker):
    n_groups = chunk_cols // _LANES
    flat = hist_len * chunk_cols
    cols_per_worker = chunk_cols * chunks_per_worker
    mesh = plsc.VectorSubcoreMesh(
        core_axis_name="c",
        subcore_axis_name="s",
        num_cores=_NUM_CORES,
        num_subcores=_NUM_SUBCORES,
    )

    @functools.partial(
        pl.kernel,
        out_type=jax.ShapeDtypeStruct((batch,), jnp.float32),
        mesh=mesh,
        scratch_types=[
            pltpu.VMEM((flat,), jnp.int32),
            pltpu.VMEM((flat,), jnp.float32),
            pltpu.VMEM((cols_per_worker,), jnp.float32),
            pltpu.SemaphoreType.DMA,
        ],
    )
    def gather_kernel(xp_hbm, p_hbm, out_hbm, idx_v, vals_v, out_v, sem):
        wid = lax.axis_index("s") * _NUM_CORES + lax.axis_index("c")
        for c in range(chunks_per_worker):
            g_chunk = wid * chunks_per_worker + c
            pltpu.sync_copy(xp_hbm.at[g_chunk], idx_v)
            pltpu.async_copy(p_hbm.at[idx_v], vals_v, sem).wait()
            for g in range(n_groups):
                def body(l, acc):
                    return acc + vals_v[pl.ds(l * chunk_cols + g * _LANES,
                                              _LANES)]
                z = lax.fori_loop(
                    0, hist_len, body, jnp.zeros((_LANES,), jnp.float32)
                )
                e = jnp.exp(-2.0 * jnp.abs(z))
                t = (1.0 - e) / (1.0 + e)
                out_v[pl.ds(c * chunk_cols + g * _LANES, _LANES)] = (
                    jnp.where(z < 0.0, -t, t)
                )
        pltpu.sync_copy(out_v, out_hbm.at[pl.ds(wid * cols_per_worker,
                                                cols_per_worker)])

    return gather_kernel


def kernel(x, table, W, b):
    batch, hist_len = x.shape
    v_rows, emb = table.shape

    p = _project_table(
        table, W.reshape(1, emb), b.reshape(1, 1).astype(jnp.float32), hist_len
    )

    chunk_cols = 128
    n_chunks = batch // chunk_cols
    chunks_per_worker = n_chunks // _NUM_WORKERS
    xp = (
        x.astype(jnp.int32)
        .reshape(n_chunks, chunk_cols, hist_len)
        .transpose(0, 2, 1)
        .reshape(n_chunks, hist_len * chunk_cols)
    )

    gather = _make_gather_kernel(hist_len, batch, chunk_cols,
                                 chunks_per_worker)
    out = gather(xp, p)
    return out.reshape(batch, 1)


# speedup vs baseline: 4.5254x; 1.4098x over previous
import functools

import jax
import jax.numpy as jnp
from jax import lax
from jax.experimental import pallas as pl
from jax.experimental.pallas import tpu as pltpu
from jax.experimental.pallas import tpu_sc as plsc

_NUM_CORES = 2
_NUM_SUBCORES = 16
_LANES = 16
_NUM_WORKERS = _NUM_CORES * _NUM_SUBCORES


def _project_body(t_ref, w_ref, b_ref, p_ref, *, inv_l):
    t = t_ref[...]
    w = w_ref[...]
    s = lax.dot_general(
        w, t, dimension_numbers=(((1,), (1,)), ((), ())),
        preferred_element_type=jnp.float32,
    )
    p_ref[...] = jnp.squeeze((s + b_ref[0, 0]) * inv_l, axis=0)


def _project_table(table, w_row, b2, hist_len):
    v_rows, emb = table.shape
    blk = 8192
    n_blk = (v_rows + blk - 1) // blk
    return pl.pallas_call(
        functools.partial(_project_body, inv_l=1.0 / float(hist_len)),
        grid=(n_blk,),
        in_specs=[
            pl.BlockSpec((blk, emb), lambda i: (i, 0)),
            pl.BlockSpec((1, emb), lambda i: (0, 0)),
            pl.BlockSpec(memory_space=pltpu.SMEM),
        ],
        out_specs=pl.BlockSpec((blk,), lambda i: (i,)),
        out_shape=jax.ShapeDtypeStruct((v_rows,), jnp.float32),
    )(table, w_row, b2)


def _make_gather_kernel(hist_len, batch, chunk_cols, chunks_per_worker):
    n_groups = chunk_cols // _LANES
    flat = hist_len * chunk_cols
    cols_per_worker = chunk_cols * chunks_per_worker
    mesh = plsc.VectorSubcoreMesh(
        core_axis_name="c",
        subcore_axis_name="s",
        num_cores=_NUM_CORES,
        num_subcores=_NUM_SUBCORES,
    )

    @functools.partial(
        pl.kernel,
        out_type=jax.ShapeDtypeStruct((batch,), jnp.float32),
        mesh=mesh,
        scratch_types=[
            pltpu.VMEM((flat,), jnp.int32),
            pltpu.VMEM((flat,), jnp.float32),
            pltpu.VMEM((cols_per_worker,), jnp.float32),
            pltpu.SemaphoreType.DMA,
        ],
    )
    def gather_kernel(xp_hbm, p_hbm, out_hbm, idx_v, vals_v, out_v, sem):
        wid = lax.axis_index("s") * _NUM_CORES + lax.axis_index("c")
        for c in range(chunks_per_worker):
            g_chunk = wid * chunks_per_worker + c
            pltpu.sync_copy(xp_hbm.at[g_chunk], idx_v)
            pltpu.async_copy(p_hbm.at[idx_v], vals_v, sem).wait()
            for g in range(n_groups):
                def body(l, acc):
                    return acc + vals_v[pl.ds(l * chunk_cols + g * _LANES,
                                              _LANES)]
                z = lax.fori_loop(
                    0, hist_len, body, jnp.zeros((_LANES,), jnp.float32)
                )
                e = jnp.exp(-2.0 * jnp.abs(z))
                t = (1.0 - e) / (1.0 + e)
                out_v[pl.ds(c * chunk_cols + g * _LANES, _LANES)] = (
                    jnp.where(z < 0.0, -t, t)
                )
        pltpu.sync_copy(out_v, out_hbm.at[pl.ds(wid * cols_per_worker,
                                                cols_per_worker)])

    return gather_kernel


def kernel(x, table, W, b):
    batch, hist_len = x.shape
    v_rows, emb = table.shape

    p = _project_table(
        table, W.reshape(1, emb), b.reshape(1, 1).astype(jnp.float32), hist_len
    )

    chunk_cols = 128
    n_chunks = batch // chunk_cols
    chunks_per_worker = n_chunks // _NUM_WORKERS
    xp = (
        x.astype(jnp.int32)
        .reshape(n_chunks, chunk_cols, hist_len)
        .transpose(0, 2, 1)
        .reshape(n_chunks, hist_len * chunk_cols)
    )

    gather = _make_gather_kernel(hist_len, batch, chunk_cols,
                                 chunks_per_worker)
    out = gather(xp, p)
    return out.reshape(batch, 1)
